# Initial kernel scaffold; baseline (speedup 1.0000x reference)
import jax
import jax.numpy as jnp
from jax.experimental import pallas as pl


def kernel(signal, window):
    raise NotImplementedError("write your pallas kernel here")


# speedup vs baseline: 1.0259x; 1.0259x over previous
import functools

import numpy as np
import jax
import jax.numpy as jnp
from jax import lax
from jax.experimental import pallas as pl
from jax.experimental.pallas import tpu as pltpu
from jax.experimental.pallas import tpu_sc as plsc

_N_FFT = 1024
_HOP = 256
_SR = 44100
_SIG_LEN = 2097252
_NB_F = 513
_N_FRAMES = 8193
_KF = 520
_B = 256
_F_PAD = 8448
_ROWS_PAD = 8320
_CHUNK = 130
_NCHUNKS = 64
_BAND = _CHUNK * _KF
_PIECES = 10
_NC, _NS = 2, 16
_NW = _NC * _NS
_WIN_D = float(_N_FFT) / _SR
_DUR = _SIG_LEN / _SR
_T_HI = _N_FRAMES * _HOP / _SR
_WT = _T_HI / _N_FRAMES
_WF = 0.5 / _NB_F
_EPS = float(np.finfo(np.float32).eps)


def _k1_body(srr, sir, trr, tir, wc, gidx_out, w_out):
    sr = srr[...]
    si = sir[...]
    tr = trr[...]
    tim = tir[...]

    w = jnp.sqrt(sr * sr + si * si) / np.float32(_NB_F)

    two_pi = np.float32(2.0 * np.pi)
    pr = sr * tr + si * tim
    pi_ = si * tr - sr * tim
    f = (jnp.arctan2(pi_, pr) / two_pi) % np.float32(1.0)

    zcol = jnp.zeros((sr.shape[0], 1), jnp.float32)
    fr = jnp.concatenate([zcol, sr[:, : _KF - 1]], axis=1)
    fi2 = jnp.concatenate([zcol, si[:, : _KF - 1]], axis=1)
    qr = sr * fr + si * fi2
    qi = si * fr - sr * fi2
    td = np.float32(0.5) - ((jnp.arctan2(qi, qr) / two_pi) % np.float32(1.0))

    t = wc[...] + td * np.float32(_WIN_D)

    inb = ((f >= np.float32(0.0)) & (f <= np.float32(0.5))
           & (t >= np.float32(0.0)) & (t <= np.float32(_T_HI)))
    w_eff = jnp.where(inb, w, np.float32(0.0))
    row = (pl.program_id(0) * _B
           + lax.broadcasted_iota(jnp.int32, (_B, _KF), 0))
    w_eff = jnp.where(row < _N_FRAMES, w_eff, np.float32(0.0))

    fi_b = jnp.clip(jnp.floor(f / np.float32(_WF)).astype(jnp.int32),
                    0, _NB_F - 1)
    ti_b = jnp.clip(jnp.floor(t / np.float32(_WT)).astype(jnp.int32),
                    0, _N_FRAMES - 1)
    gidx = ti_b * np.int32(_KF) + fi_b
    gidx_out[...] = jnp.where(row < _N_FRAMES, gidx, np.int32(-1))
    w_out[...] = w_eff


def _k1(srr, sir, trr, tir, wc):
    bspec = pl.BlockSpec((_B, _KF), lambda g: (g, 0))
    return pl.pallas_call(
        _k1_body,
        grid=(_F_PAD // _B,),
        in_specs=[bspec] * 4 + [pl.BlockSpec((_B, 1), lambda g: (g, 0))],
        out_specs=[bspec, bspec],
        out_shape=[jax.ShapeDtypeStruct((_F_PAD, _KF), jnp.int32),
                   jax.ShapeDtypeStruct((_F_PAD, _KF), jnp.float32)],
    )(srr, sir, trr, tir, wc)


_STRIPE_F = 128
_NSTRIPES = _F_PAD // _STRIPE_F
_STRIPE_W = _STRIPE_F * _KF
_PIECE_W = 16 * _KF


def _sc_hist(gidx_flat, w_flat):
    mesh = plsc.VectorSubcoreMesh(core_axis_name="c", subcore_axis_name="s")

    @functools.partial(
        pl.kernel, mesh=mesh,
        compiler_params=pltpu.CompilerParams(needs_layout_passes=False),
        out_type=jax.ShapeDtypeStruct((_ROWS_PAD * _KF,), jnp.float32),
        scratch_types=[
            pltpu.VMEM((_BAND,), jnp.float32),
            pltpu.VMEM((_PIECE_W,), jnp.int32),
            pltpu.VMEM((_PIECE_W,), jnp.float32),
        ],
    )
    def k(gidx_hbm, w_hbm, out_hbm, band, gbuf, wbuf):
        wid = lax.axis_index("s") * _NC + lax.axis_index("c")
        for kk in range(2):
            cid = wid + _NW * kk
            c0w = cid * _BAND

            def zbody(i, carry):
                band[pl.ds(i * 16, 16)] = jnp.zeros((16,), jnp.float32)
                return carry

            lax.fori_loop(0, _BAND // 16, zbody, 0)

            s_lo = jnp.maximum(cid * _CHUNK - 8, 0) // _STRIPE_F
            for st in range(3):
                for p in range(8):
                    start = (s_lo + st) * _STRIPE_W + p * _PIECE_W
                    pltpu.sync_copy(gidx_hbm.at[pl.ds(start, _PIECE_W)],
                                    gbuf)
                    pltpu.sync_copy(w_hbm.at[pl.ds(start, _PIECE_W)], wbuf)

                    def sbody(i, carry):
                        base = i * 16
                        kvec = gbuf[pl.ds(base, 16)] - c0w
                        wvec = wbuf[pl.ds(base, 16)]
                        maskv = (kvec >= 0) & (kvec < _BAND)
                        kc = jnp.clip(kvec, 0, _BAND - 1)
                        plsc.addupdate_scatter(band, [kc], wvec, mask=maskv)
                        return carry

                    lax.fori_loop(0, _PIECE_W // 16, sbody, 0)

            pltpu.sync_copy(band, out_hbm.at[pl.ds(c0w, _BAND)])

    return k(gidx_flat, w_flat)


def _k3_body(h_ref, o_ref):
    x = h_ref[...]
    o_ref[...] = np.float32(20.0) * jnp.log10(
        jnp.maximum(np.float32(1e-06), x))


def _k3(hist2d):
    return pl.pallas_call(
        _k3_body,
        grid=(_ROWS_PAD // 128,),
        in_specs=[pl.BlockSpec((128, _KF), lambda g: (g, 0))],
        out_specs=pl.BlockSpec((128, _KF), lambda g: (g, 0)),
        out_shape=jax.ShapeDtypeStruct((_ROWS_PAD, _KF), jnp.float32),
    )(hist2d)


def _stft_fr(x, window):
    pad = _N_FFT // 2
    xp = jnp.pad(x, (pad, pad), mode="reflect")
    n_frames = 1 + (xp.shape[0] - _N_FFT) // _HOP
    idx = (jnp.arange(n_frames)[:, None] * _HOP
           + jnp.arange(_N_FFT)[None, :])
    frames = xp[idx] * window[None, :]
    return jnp.fft.rfft(frames, axis=1)


def _pad2(x):
    return jnp.pad(x, ((0, _F_PAD - _N_FRAMES), (0, _KF - _NB_F)))


def kernel(signal, window):
    spec = _stft_fr(signal, window)
    ts = jnp.roll(signal, 1).at[0].set(0.0)
    spec_ts = _stft_fr(ts, window)

    sr = _pad2(jnp.real(spec))
    si = _pad2(jnp.imag(spec))
    tr = _pad2(jnp.real(spec_ts))
    tim = _pad2(jnp.imag(spec_ts))

    wst = jnp.arange(0.0, _DUR, _HOP / _SR)
    wc = wst + _WIN_D / 2 + _EPS
    wc = jnp.pad(wc, (0, _F_PAD - _N_FRAMES)).reshape(_F_PAD, 1)

    gidx, w = _k1(sr, si, tr, tim, wc)

    def stripe_perm(x):
        y = x.reshape(_NSTRIPES, 16, 8, _KF).transpose(0, 2, 1, 3)
        return y.reshape(_NSTRIPES, _STRIPE_F, _KF).transpose(0, 2, 1)

    gidx_p = stripe_perm(gidx).reshape(-1)
    w_p = stripe_perm(w).reshape(-1)
    hist_flat = _sc_hist(gidx_p, w_p)
    out_p = _k3(hist_flat.reshape(_ROWS_PAD, _KF))
    return out_p[:_N_FRAMES, :_NB_F].T
